# Initial kernel scaffold; baseline (speedup 1.0000x reference)
import jax
import jax.numpy as jnp
from jax.experimental import pallas as pl


def kernel(boxes, scores):
    raise NotImplementedError("write your pallas kernel here")


# speedup vs baseline: 4.6083x; 4.6083x over previous
import jax
import jax.numpy as jnp
from jax import lax
from jax.experimental import pallas as pl
from jax.experimental.pallas import tpu as pltpu
from jax.experimental.pallas import tpu_sc as plsc

_N = 5000
_NPAD = 5120
_MAX_DET = 300
_OUTPAD = 304
_SIGMA = 0.5
_THRESH = 0.001
_NSUB = 16
_NCORE = 2
_PER = _NPAD // _NSUB
_CH = _PER // 16
_BOARD = 256


def _nms_body(x1h, y1h, x2h, y2h, sch, outh,
              x1, y1, x2, y2, ar, msc, stage, shv, loc, outv):
    sid = lax.axis_index("s")
    cid = lax.axis_index("c")
    base = sid * _PER
    is_out_tile = (sid == 0) & (cid == 0)

    pltpu.sync_copy(x1h, x1)
    pltpu.sync_copy(y1h, y1)
    pltpu.sync_copy(x2h, x2)
    pltpu.sync_copy(y2h, y2)
    pltpu.sync_copy(sch.at[pl.ds(base, _PER)], msc)

    li = lax.iota(jnp.int32, 16)
    lif = li.astype(jnp.float32)
    base_f = base.astype(jnp.float32)

    def _area(j, carry):
        xs = x1[pl.ds(j * 16, 16)]
        ys = y1[pl.ds(j * 16, 16)]
        a = (x2[pl.ds(j * 16, 16)] - xs) * (y2[pl.ds(j * 16, 16)] - ys)
        ar[pl.ds(j * 16, 16)] = a
        return carry

    lax.fori_loop(0, _NPAD // 16, _area, 0)

    @pl.when(is_out_tile)
    def _():
        for j in range(_OUTPAD // 16):
            outv[pl.ds(j * 16, 16)] = jnp.zeros((16,), jnp.float32)

    lane0 = li == 0

    def _step(t, carry):
        bestv = msc[pl.ds(0, 16)]
        bestif = lif + base_f
        for j in range(1, _CH):
            v = msc[pl.ds(j * 16, 16)]
            idxf = lif + (base_f + jnp.float32(j * 16))
            gt = v > bestv
            bestif = jnp.where(gt, idxf, bestif)
            bestv = jnp.maximum(v, bestv)
        mv = jnp.max(bestv)
        mi = jnp.min(jnp.where(bestv == mv, bestif, jnp.float32(1e9)))

        pbase = (t % 2) * _BOARD
        tagval = (t + 1).astype(jnp.float32)
        stage[pl.ds(0, 16)] = jnp.where(lane0, mv, jnp.where(li == 1, mi,
                                                             jnp.float32(0.0)))
        pltpu.sync_copy(stage.at[pl.ds(0, 8)],
                        shv.at[pl.ds(pbase + sid * 16, 8)])
        stage[pl.ds(0, 16)] = jnp.full((16,), tagval, jnp.float32)
        pltpu.sync_copy(stage.at[pl.ds(0, 8)],
                        shv.at[pl.ds(pbase + sid * 16 + 8, 8)])

        def poll_cond(c):
            return (c[1] == 0) & (c[0] < 10000)

        def poll_body(c):
            pltpu.sync_copy(shv.at[pl.ds(pbase, _BOARD)], loc)
            tags = plsc.load_gather(loc, [li * 16 + 8])
            good = jnp.min(jnp.where(tags == tagval, 1, 0))
            return (c[0] + 1, good)

        lax.while_loop(poll_cond, poll_body, (jnp.int32(0), jnp.int32(0)))
        vals = plsc.load_gather(loc, [li * 16])
        idxs = plsc.load_gather(loc, [li * 16 + 1])
        gv = jnp.max(vals)
        gif = jnp.min(jnp.where(vals == gv, idxs, jnp.float32(1e9)))
        g = gif.astype(jnp.int32)

        @pl.when(is_out_tile)
        def _():
            plsc.store_scatter(outv, [jnp.full((16,), t, jnp.int32)],
                               jnp.full((16,), gv, jnp.float32), mask=lane0)

        gsplat = jnp.full((16,), g, jnp.int32)
        xa = plsc.load_gather(x1, [gsplat])
        ya = plsc.load_gather(y1, [gsplat])
        xb = plsc.load_gather(x2, [gsplat])
        yb = plsc.load_gather(y2, [gsplat])
        ag = plsc.load_gather(ar, [gsplat])
        for j in range(_CH):
            o = base + j * 16
            sv = msc[pl.ds(j * 16, 16)]
            bx1 = x1[pl.ds(o, 16)]
            by1 = y1[pl.ds(o, 16)]
            bx2 = x2[pl.ds(o, 16)]
            by2 = y2[pl.ds(o, 16)]
            av = ar[pl.ds(o, 16)]
            ltx = jnp.maximum(bx1, xa)
            lty = jnp.maximum(by1, ya)
            rbx = jnp.minimum(bx2, xb)
            rby = jnp.minimum(by2, yb)
            w = jnp.maximum(rbx - ltx, jnp.float32(0.0))
            h = jnp.maximum(rby - lty, jnp.float32(0.0))
            inter = w * h
            union = av + ag - inter
            iou = inter / (union + jnp.float32(1e-9))
            dec = jnp.exp(iou * iou * jnp.float32(-1.0 / _SIGMA))
            msc[pl.ds(j * 16, 16)] = sv * dec

        @pl.when((g >= base) & (g < base + _PER))
        def _():
            plsc.store_scatter(msc, [jnp.full((16,), g - base, jnp.int32)],
                               jnp.full((16,), -1.0, jnp.float32), mask=lane0)

        return carry

    lax.fori_loop(0, _MAX_DET, _step, 0)

    @pl.when(is_out_tile)
    def _():
        for j in range(_OUTPAD // 16):
            v = outv[pl.ds(j * 16, 16)]
            outv[pl.ds(j * 16, 16)] = jnp.where(v < jnp.float32(_THRESH),
                                                jnp.float32(0.0), v)
        pltpu.sync_copy(outv.at[pl.ds(0, _MAX_DET)], outh)


def _make_nms():
    mesh = plsc.VectorSubcoreMesh(core_axis_name="c", subcore_axis_name="s",
                                  num_cores=_NCORE, num_subcores=_NSUB)
    return pl.kernel(
        _nms_body,
        out_type=jax.ShapeDtypeStruct((_MAX_DET,), jnp.float32),
        mesh=mesh,
        scratch_types=[
            pltpu.VMEM((_NPAD,), jnp.float32),
            pltpu.VMEM((_NPAD,), jnp.float32),
            pltpu.VMEM((_NPAD,), jnp.float32),
            pltpu.VMEM((_NPAD,), jnp.float32),
            pltpu.VMEM((_NPAD,), jnp.float32),
            pltpu.VMEM((_PER,), jnp.float32),
            pltpu.VMEM((16,), jnp.float32),
            pltpu.VMEM_SHARED((2 * _BOARD,), jnp.float32),
            pltpu.VMEM((_BOARD,), jnp.float32),
            pltpu.VMEM((_OUTPAD,), jnp.float32),
        ],
        compiler_params=pltpu.CompilerParams(needs_layout_passes=False),
    )


@jax.jit
def kernel(boxes, scores):
    bp = jnp.pad(boxes, ((0, _NPAD - _N), (0, 0)))
    sp = jnp.pad(scores, (0, _NPAD - _N), constant_values=-1.0)
    x1 = bp[:, 0]
    y1 = bp[:, 1]
    x2 = bp[:, 2]
    y2 = bp[:, 3]
    return _make_nms()(x1, y1, x2, y2, sp)
